# Initial kernel scaffold; baseline (speedup 1.0000x reference)
import jax
import jax.numpy as jnp
from jax.experimental import pallas as pl


def kernel(node_feat, edge_feat, edge_index, graph_ids, params):
    raise NotImplementedError("write your pallas kernel here")


# speedup vs baseline: 1.2591x; 1.2591x over previous
import functools

import jax
import jax.numpy as jnp
from jax import lax
from jax.experimental import pallas as pl
from jax.experimental.pallas import tpu as pltpu
from jax.experimental.pallas import tpu_sc as plsc

_N = 10000
_E = 320000
_H = 128
_RV = 10
_EMB = _H - _RV
_L = 3
_NG = 64
_NP = 10240
_NW = 32
_NR = 80
_RNG = 128
_EW = _E // _NW
_GE = 32
_CAPG = 314
_CE_BIN = 2000
_F32 = jnp.float32
_I32 = jnp.int32
_HI = lax.Precision.HIGHEST

@functools.cache
def _sc_mesh():
    return plsc.VectorSubcoreMesh(core_axis_name="c", subcore_axis_name="s")


def _wid():
    return lax.axis_index("s") * 2 + lax.axis_index("c")


def _splat(x):
    return jnp.full((16,), x, _I32)


def _extract0(lane, v):
    return jnp.sum(jnp.where(lane == 0, v, 0))


def _bin_body(src_hbm, dst_hbm, bkt_hbm, nreal_hbm,
              stage, cursor, grpcnt, realc, srcb, dstb):
    w = _wid()
    lane = lax.iota(_I32, 16)
    zeros16 = jnp.zeros((16,), _I32)
    ones16 = jnp.ones((16,), _I32)
    m3 = lane < 3
    m1 = lane < 1
    for q in range(_NR // 16):
        cursor[pl.ds(q * 16, 16)] = zeros16
        grpcnt[pl.ds(q * 16, 16)] = zeros16
        realc[pl.ds(q * 16, 16)] = zeros16

    for ci in range(_EW // _CE_BIN):
        base = w * _EW + ci * _CE_BIN
        pltpu.sync_copy(src_hbm.at[pl.ds(base, _CE_BIN)], srcb)
        pltpu.sync_copy(dst_hbm.at[pl.ds(base, _CE_BIN)], dstb)

        def ebody(e, _, base=base):
            esp = _splat(e)
            dv = plsc.load_gather(dstb, [esp])
            sv = plsc.load_gather(srcb, [esp])
            rv = lax.shift_right_logical(dv, 7)
            dl = jnp.bitwise_and(dv, 127)
            cv = plsc.load_gather(cursor, [rv])
            eidv = _splat(base + e)
            vals = jnp.where(lane == 0, eidv, jnp.where(lane == 1, sv, dl))
            plsc.store_scatter(stage, [rv * 96 + cv * 3 + lane], vals, mask=m3)
            plsc.addupdate_scatter(cursor, [rv], ones16, mask=m1)
            plsc.addupdate_scatter(realc, [rv], ones16, mask=m1)
            c_s = _extract0(lane, cv)
            r_s = _extract0(lane, rv)

            @pl.when(c_s == _GE - 1)
            def _flush():
                rsp = _splat(r_s)
                g_s = _extract0(lane, plsc.load_gather(grpcnt, [rsp]))
                pltpu.sync_copy(
                    stage.at[pl.ds(r_s * 96, 96)],
                    bkt_hbm.at[pl.ds((r_s * _NW + w) * (_CAPG * 96)
                                     + g_s * 96, 96)])
                plsc.store_scatter(cursor, [rsp], zeros16, mask=m1)
                plsc.addupdate_scatter(grpcnt, [rsp], ones16, mask=m1)
            return 0

        lax.fori_loop(0, _CE_BIN, ebody, 0)

    padvals = jnp.where(lane == 2, _splat(_RNG), zeros16)

    def rbody(r, _):
        rsp = _splat(r)
        c_s = _extract0(lane, plsc.load_gather(cursor, [rsp]))
        g_s = _extract0(lane, plsc.load_gather(grpcnt, [rsp]))

        @pl.when(c_s > 0)
        def _():
            def pbody(q, _):
                plsc.store_scatter(stage, [r * 96 + q * 3 + lane], padvals,
                                   mask=m3)
                return 0
            lax.fori_loop(c_s, _GE, pbody, 0)
            pltpu.sync_copy(
                stage.at[pl.ds(r * 96, 96)],
                bkt_hbm.at[pl.ds((r * _NW + w) * (_CAPG * 96)
                                 + g_s * 96, 96)])

        ng2 = g_s + jnp.where(c_s > 0, 1, 0)

        @pl.when(jnp.bitwise_and(ng2, 1) == 1)
        def _():
            def pb2(q, _):
                plsc.store_scatter(stage, [r * 96 + q * 3 + lane], padvals,
                                   mask=m3)
                return 0
            lax.fori_loop(0, _GE, pb2, 0)
            pltpu.sync_copy(
                stage.at[pl.ds(r * 96, 96)],
                bkt_hbm.at[pl.ds((r * _NW + w) * (_CAPG * 96)
                                 + ng2 * 96, 96)])
        return 0

    lax.fori_loop(0, _NR, rbody, 0)
    pltpu.sync_copy(realc, nreal_hbm.at[pl.ds(w * _NR, _NR)])


def _edge_body(bkt_hbm, nreal_hbm, a_hbm, b_hbm, c_hbm,
               out_hbm, deg_hbm,
               acc_s, acc_q, acc_mx, acc_mn, acc_d, bbuf, gbuf, sidx, eidx,
               arows, crows, nrbuf, sem_a, sem_c):
    w = _wid()
    lane = lax.iota(_I32, 16)
    zf = jnp.zeros((16,), _F32)
    negf = jnp.full((16,), -3.0e38, _F32)
    posf = jnp.full((16,), 3.0e38, _F32)
    onesf = jnp.ones((16,), _F32)
    m1 = lane < 1
    pltpu.sync_copy(nreal_hbm, nrbuf)

    def do_range(r):
        def ib(i, _):
            isp = _splat(i)
            for j in range(8):
                col = j * 16 + lane
                plsc.store_scatter(acc_s, [isp, col], zf)
                plsc.store_scatter(acc_q, [isp, col], zf)
                plsc.store_scatter(acc_mx, [isp, col], negf)
                plsc.store_scatter(acc_mn, [isp, col], posf)
            return 0
        lax.fori_loop(0, 136, ib, 0)
        for q in range(9):
            acc_d[pl.ds(q * 16, 16)] = zf
        pltpu.sync_copy(b_hbm.at[pl.ds(r * _RNG, _RNG), :],
                        bbuf.at[pl.ds(0, _RNG), :])

        def swbody(sw, _):
            nv = plsc.load_gather(nrbuf, [_splat(sw) * _NR + _splat(r)])
            n_s = _extract0(lane, nv)
            nch = lax.shift_right_logical(n_s + 63, 6)

            def chbody(k, _):
                pltpu.sync_copy(
                    bkt_hbm.at[pl.ds((r * _NW + sw) * (_CAPG * 96)
                                     + k * 192, 192)], gbuf)
                for q in range(4):
                    i3 = (lane + q * 16) * 3
                    eidx[pl.ds(q * 16, 16)] = plsc.load_gather(gbuf, [i3])
                    sidx[pl.ds(q * 16, 16)] = plsc.load_gather(gbuf, [i3 + 1])
                cp_a = pltpu.async_copy(a_hbm.at[sidx], arows, sem_a)
                cp_c = pltpu.async_copy(c_hbm.at[eidx], crows, sem_c)
                cp_a.wait()
                cp_c.wait()
                e_hi = jnp.minimum(64, n_s - k * 64)

                def ebody(e, _):
                    dlsp = plsc.load_gather(gbuf, [_splat(e * 3 + 2)])
                    esp = _splat(e)
                    plsc.addupdate_scatter(acc_d, [dlsp], onesf, mask=m1)
                    for j in range(8):
                        col = j * 16 + lane
                        av = plsc.load_gather(arows, [esp, col])
                        cv = plsc.load_gather(crows, [esp, col])
                        bv = plsc.load_gather(bbuf, [dlsp, col])
                        m = av + cv + bv
                        plsc.addupdate_scatter(acc_s, [dlsp, col], m)
                        plsc.addupdate_scatter(acc_q, [dlsp, col], m * m)
                        mx = plsc.load_gather(acc_mx, [dlsp, col])
                        plsc.store_scatter(acc_mx, [dlsp, col],
                                           jnp.maximum(mx, m))
                        mn = plsc.load_gather(acc_mn, [dlsp, col])
                        plsc.store_scatter(acc_mn, [dlsp, col],
                                           jnp.minimum(mn, m))
                    return 0

                lax.fori_loop(0, e_hi, ebody, 0)
                return 0

            lax.fori_loop(0, nch, chbody, 0)
            return 0

        lax.fori_loop(0, _NW, swbody, 0)
        pltpu.sync_copy(acc_s.at[pl.ds(0, _RNG), :],
                        out_hbm.at[pl.ds(r * _RNG, _RNG), pl.ds(0, 128)])
        pltpu.sync_copy(acc_q.at[pl.ds(0, _RNG), :],
                        out_hbm.at[pl.ds(r * _RNG, _RNG), pl.ds(128, 128)])
        pltpu.sync_copy(acc_mx.at[pl.ds(0, _RNG), :],
                        out_hbm.at[pl.ds(r * _RNG, _RNG), pl.ds(256, 128)])
        pltpu.sync_copy(acc_mn.at[pl.ds(0, _RNG), :],
                        out_hbm.at[pl.ds(r * _RNG, _RNG), pl.ds(384, 128)])
        pltpu.sync_copy(acc_d.at[pl.ds(0, _RNG)],
                        deg_hbm.at[pl.ds(r * _RNG, _RNG)])

    do_range(w)
    do_range(w + 32)
    r3 = w + 64

    @pl.when(r3 < _NR)
    def _():
        do_range(r3)


def _gmax_body(h_hbm, gid_hbm, gmp_hbm, acc_g, gidb, hbuf):
    w = _wid()
    lane = lax.iota(_I32, 16)
    negf = jnp.full((16,), -3.0e38, _F32)

    def ib(i, _):
        isp = _splat(i)
        for j in range(8):
            plsc.store_scatter(acc_g, [isp, j * 16 + lane], negf)
        return 0
    lax.fori_loop(0, 72, ib, 0)
    pltpu.sync_copy(gid_hbm.at[pl.ds(w * 320, 320)], gidb)
    for ci in range(10):
        pltpu.sync_copy(h_hbm.at[pl.ds(w * 320 + ci * 32, 32), :], hbuf)

        def vb(v, _, ci=ci):
            gsp = plsc.load_gather(gidb, [_splat(ci * 32 + v)])
            vsp = _splat(v)
            for j in range(8):
                col = j * 16 + lane
                hv = plsc.load_gather(hbuf, [vsp, col])
                og = plsc.load_gather(acc_g, [gsp, col])
                plsc.store_scatter(acc_g, [gsp, col], jnp.maximum(og, hv))
            return 0

        lax.fori_loop(0, 32, vb, 0)
    pltpu.sync_copy(acc_g.at[pl.ds(0, _NG), :], gmp_hbm.at[w])


def _enc_body(nf_ref, t_ref, rx_ref, o_ref):
    f = nf_ref[...]
    cols = lax.broadcasted_iota(_I32, (512, 576), 1)
    oh = jnp.zeros((512, 576), _F32)
    for i in range(9):
        oh = oh + (cols == (f[:, i:i + 1] + 64 * i)).astype(_F32)
    o_ref[...] = (jnp.dot(oh, t_ref[...], preferred_element_type=_F32,
                          precision=_HI) + rx_ref[...])


def _cenc_body(ef_ref, re_ref, bf_ref, w3_ref, wrv_ref,
               c0_ref, c1_ref, c2_ref):
    u = jnp.dot(bf_ref[...], w3_ref[...], preferred_element_type=_F32,
                precision=_HI)
    f = ef_ref[...]
    cols = lax.broadcasted_iota(_I32, (1280, 24), 1)
    oh = jnp.zeros((1280, 24), _F32)
    for i in range(3):
        oh = oh + (cols == (f[:, i:i + 1] + 8 * i)).astype(_F32)
    cc = (jnp.dot(oh, u, preferred_element_type=_F32, precision=_HI)
          + jnp.dot(re_ref[...], wrv_ref[...], preferred_element_type=_F32,
                    precision=_HI))
    c0_ref[...] = cc[:, 0:128]
    c1_ref[...] = cc[:, 128:256]
    c2_ref[...] = cc[:, 256:384]


def _pre_body(h_ref, w1_ref, w2_ref, b_ref, a_ref, bo_ref):
    h = h_ref[...]
    a_ref[...] = (jnp.dot(h, w1_ref[...], preferred_element_type=_F32,
                          precision=_HI) + b_ref[...])
    bo_ref[...] = jnp.dot(h, w2_ref[...], preferred_element_type=_F32,
                          precision=_HI)


def _post_body(h_ref, s_ref, d_ref, wp_ref, bp_ref, o_ref):
    h = h_ref[...]
    sall = s_ref[...]
    d = d_ref[...]
    cnt = jnp.maximum(d, 1.0)
    logd = jnp.log(d + 1.0)
    att = 1.0 / jnp.where(logd > 0, logd, 1.0)
    mean = sall[:, 0:128] / cnt
    sq = sall[:, 128:256] / cnt
    std = jnp.sqrt(jax.nn.relu(sq - mean * mean) + 1e-5)
    mx = jnp.where(d > 0, sall[:, 256:384], 0.0)
    mn = jnp.where(d > 0, sall[:, 384:512], 0.0)
    wp = wp_ref[...]
    feats = (h, mean, mx, mn, std,
             mean * logd, mx * logd, mn * logd, std * logd,
             mean * att, mx * att, mn * att, std * att)
    acc = h + bp_ref[...]
    for i, x in enumerate(feats):
        acc = acc + jnp.dot(x, wp[i], preferred_element_type=_F32,
                            precision=_HI)
    o_ref[...] = acc


def _read_body(h_ref, gid_ref, gmp_ref, w1_ref, b1_ref, w2_ref, b2_ref,
               o_ref, gsum, gcnt):
    i = pl.program_id(0)

    @pl.when(i == 0)
    def _():
        gsum[...] = jnp.zeros((_NG, 128), _F32)
        gcnt[...] = jnp.zeros((_NG, 128), _F32)

    gid = gid_ref[0, 0, :]
    m = (gid[None, :] == lax.broadcasted_iota(_I32, (_NG, 400), 0)).astype(
        _F32)
    gsum[...] += jnp.dot(m, h_ref[...], preferred_element_type=_F32,
                         precision=_HI)
    gcnt[...] += jnp.sum(m, axis=1, keepdims=True)

    @pl.when(i == 24)
    def _():
        gs = gsum[...]
        gc = gcnt[...]
        gmx = jnp.where(gc > 0, jnp.max(gmp_ref[...], axis=0), 0.0)
        gmean = gs / jnp.maximum(gc, 1.0)
        w1 = w1_ref[...]
        hid = jax.nn.relu(
            jnp.dot(gs, w1[0], preferred_element_type=_F32, precision=_HI)
            + jnp.dot(gmean, w1[1], preferred_element_type=_F32,
                      precision=_HI)
            + jnp.dot(gmx, w1[2], preferred_element_type=_F32, precision=_HI)
            + b1_ref[...])
        o_ref[...] = (jnp.dot(hid, w2_ref[...], preferred_element_type=_F32,
                              precision=_HI) + b2_ref[...])


def kernel(node_feat, edge_feat, edge_index, graph_ids, params):
    kr = jax.random.key(42)
    k1, k2 = jax.random.split(kr)
    rand_x = jax.random.normal(k1, (_N, _RV), _F32)
    rand_e = jax.random.normal(k2, (_E, _RV), _F32)

    tpad = jnp.pad(params['atom_tables'].reshape(576, _EMB),
                   ((0, 0), (0, _RV)))
    rxpad = jnp.pad(rand_x, ((0, _NP - _N), (_EMB, 0)))
    bondflat = jnp.pad(params['bond_tables'].reshape(24, _EMB),
                       ((0, 0), (0, 2)))
    w3cat = jnp.pad(
        jnp.concatenate([params['W_pre'][l][256:256 + _EMB] for l in
                         range(_L)], axis=1), ((0, 2), (0, 0)))
    wrvcat = jnp.pad(
        jnp.concatenate([params['W_pre'][l][256 + _EMB:384] for l in
                         range(_L)], axis=1), ((0, 6), (0, 0)))
    re16 = jnp.pad(rand_e, ((0, 0), (0, 6)))
    gidp = jnp.concatenate([graph_ids,
                            jnp.full((_NP - _N,), _NG, _I32)])
    src = edge_index[0]
    dst = edge_index[1]

    h = pl.pallas_call(
        _enc_body,
        grid=(20,),
        in_specs=[pl.BlockSpec((512, 9), lambda i: (i, 0)),
                  pl.BlockSpec((576, 128), lambda i: (0, 0)),
                  pl.BlockSpec((512, 128), lambda i: (i, 0))],
        out_specs=pl.BlockSpec((512, 128), lambda i: (i, 0)),
        out_shape=jax.ShapeDtypeStruct((_NP, 128), _F32),
    )(node_feat, tpad, rxpad)

    c0, c1, c2 = pl.pallas_call(
        _cenc_body,
        grid=(250,),
        in_specs=[pl.BlockSpec((1280, 3), lambda i: (i, 0)),
                  pl.BlockSpec((1280, 16), lambda i: (i, 0)),
                  pl.BlockSpec((24, 120), lambda i: (0, 0)),
                  pl.BlockSpec((120, 384), lambda i: (0, 0)),
                  pl.BlockSpec((16, 384), lambda i: (0, 0))],
        out_specs=[pl.BlockSpec((1280, 128), lambda i: (i, 0))] * 3,
        out_shape=[jax.ShapeDtypeStruct((_E, 128), _F32)] * 3,
    )(edge_feat, re16, bondflat, w3cat, wrvcat)
    c_layers = (c0, c1, c2)

    bin_k = pl.kernel(
        _bin_body,
        out_type=(jax.ShapeDtypeStruct((_NR * _NW * _CAPG * 96,), _I32),
                  jax.ShapeDtypeStruct((_NW * _NR,), _I32)),
        mesh=_sc_mesh(),
        compiler_params=pltpu.CompilerParams(needs_layout_passes=False),
        scratch_types=[pltpu.VMEM((_NR * 96,), _I32),
                       pltpu.VMEM((_NR,), _I32),
                       pltpu.VMEM((_NR,), _I32),
                       pltpu.VMEM((_NR,), _I32),
                       pltpu.VMEM((_CE_BIN,), _I32),
                       pltpu.VMEM((_CE_BIN,), _I32)],
    )
    bkt, nreal = bin_k(src, dst)

    edge_k = pl.kernel(
        _edge_body,
        out_type=(jax.ShapeDtypeStruct((_NP, 512), _F32),
                  jax.ShapeDtypeStruct((_NP,), _F32)),
        mesh=_sc_mesh(),
        compiler_params=pltpu.CompilerParams(needs_layout_passes=False),
        scratch_types=[pltpu.VMEM((136, 128), _F32),
                       pltpu.VMEM((136, 128), _F32),
                       pltpu.VMEM((136, 128), _F32),
                       pltpu.VMEM((136, 128), _F32),
                       pltpu.VMEM((144,), _F32),
                       pltpu.VMEM((136, 128), _F32),
                       pltpu.VMEM((192,), _I32),
                       pltpu.VMEM((64,), _I32),
                       pltpu.VMEM((64,), _I32),
                       pltpu.VMEM((64, 128), _F32),
                       pltpu.VMEM((64, 128), _F32),
                       pltpu.VMEM((_NW * _NR,), _I32),
                       pltpu.SemaphoreType.DMA,
                       pltpu.SemaphoreType.DMA],
    )

    wpost = jnp.stack([params['W_post'][l].reshape(13, 128, 128)
                       for l in range(_L)])
    deg = None
    for l in range(_L):
        a_arr, b_arr = pl.pallas_call(
            _pre_body,
            grid=(20,),
            in_specs=[pl.BlockSpec((512, 128), lambda i: (i, 0)),
                      pl.BlockSpec((128, 128), lambda i: (0, 0)),
                      pl.BlockSpec((128, 128), lambda i: (0, 0)),
                      pl.BlockSpec((1, 128), lambda i: (0, 0))],
            out_specs=[pl.BlockSpec((512, 128), lambda i: (i, 0))] * 2,
            out_shape=[jax.ShapeDtypeStruct((_NP, 128), _F32)] * 2,
        )(h, params['W_pre'][l][0:128], params['W_pre'][l][128:256],
          params['b_pre'][l].reshape(1, 128))

        out_agg, deg_l = edge_k(bkt, nreal, a_arr, b_arr, c_layers[l])
        if deg is None:
            deg = deg_l

        h = pl.pallas_call(
            _post_body,
            grid=(20,),
            in_specs=[pl.BlockSpec((512, 128), lambda i: (i, 0)),
                      pl.BlockSpec((512, 512), lambda i: (i, 0)),
                      pl.BlockSpec((512, 1), lambda i: (i, 0)),
                      pl.BlockSpec((13, 128, 128), lambda i: (0, 0, 0)),
                      pl.BlockSpec((1, 128), lambda i: (0, 0))],
            out_specs=pl.BlockSpec((512, 128), lambda i: (i, 0)),
            out_shape=jax.ShapeDtypeStruct((_NP, 128), _F32),
        )(h, out_agg, deg.reshape(_NP, 1), wpost[l],
          params['b_post'][l].reshape(1, 128))

    gmax_k = pl.kernel(
        _gmax_body,
        out_type=jax.ShapeDtypeStruct((_NW, _NG, 128), _F32),
        mesh=_sc_mesh(),
        compiler_params=pltpu.CompilerParams(needs_layout_passes=False),
        scratch_types=[pltpu.VMEM((72, 128), _F32),
                       pltpu.VMEM((320,), _I32),
                       pltpu.VMEM((32, 128), _F32)],
    )
    gmp = gmax_k(h, gidp)

    out8 = pl.pallas_call(
        _read_body,
        grid=(25,),
        in_specs=[pl.BlockSpec((400, 128), lambda i: (i, 0)),
                  pl.BlockSpec((1, 1, 400), lambda i: (i, 0, 0)),
                  pl.BlockSpec((_NW, _NG, 128), lambda i: (0, 0, 0)),
                  pl.BlockSpec((3, 128, 128), lambda i: (0, 0, 0)),
                  pl.BlockSpec((1, 128), lambda i: (0, 0)),
                  pl.BlockSpec((128, 8), lambda i: (0, 0)),
                  pl.BlockSpec((1, 8), lambda i: (0, 0))],
        out_specs=pl.BlockSpec((_NG, 8), lambda i: (0, 0)),
        out_shape=jax.ShapeDtypeStruct((_NG, 8), _F32),
        scratch_shapes=[pltpu.VMEM((_NG, 128), _F32),
                        pltpu.VMEM((_NG, 128), _F32)],
    )(h[:_N], gidp[:_N].reshape(25, 1, 400),
      gmp,
      params['W_r1'].reshape(3, 128, 128),
      params['b_r1'].reshape(1, 128),
      jnp.pad(params['W_r2'], ((0, 0), (0, 7))),
      jnp.pad(params['b_r2'], (0, 7)).reshape(1, 8))

    return out8[:, 0:1]
